# baseline (speedup 1.0000x reference)
import functools

import jax
import jax.numpy as jnp
from jax import lax
from jax.experimental import pallas as pl
from jax.experimental.pallas import tpu as pltpu, tpu_sc as plsc

N = 262144
NCOL = 66
OCOL = 13
NUMERIC = 10
BLOCKS = ((10, 18), (18, 34), (34, 66))

NC = 2
NS = 16
L = 16
NW = NC * NS
ROWS_PER_W = N // NW
CH = 512
NCHUNK = ROWS_PER_W // CH

_mesh = plsc.VectorSubcoreMesh(core_axis_name="c", subcore_axis_name="s")


@functools.partial(
    pl.kernel,
    out_type=jax.ShapeDtypeStruct((N, OCOL), jnp.float32),
    mesh=_mesh,
    compiler_params=pltpu.CompilerParams(needs_layout_passes=False),
    scratch_types=[
        pltpu.VMEM((CH, NCOL), jnp.float32),
        pltpu.VMEM((CH, OCOL), jnp.float32),
    ],
)
def _onehot_to_numeric(x_hbm, out_hbm, inbuf, outbuf):
    wid = lax.axis_index("s") * NC + lax.axis_index("c")
    base_row = wid * ROWS_PER_W
    lane = lax.iota(jnp.int32, L)

    def group_body(g, carry):
        lr = g * L + lane
        for c in range(NUMERIC):
            cvec = jnp.full((L,), c, jnp.int32)
            v = plsc.load_gather(inbuf, [lr, cvec])
            plsc.store_scatter(outbuf, [lr, cvec], v)
        for slot, (s, e) in enumerate(BLOCKS):
            m = plsc.load_gather(inbuf, [lr, jnp.full((L,), s, jnp.int32)])
            a = jnp.zeros((L,), jnp.float32)
            for j in range(1, e - s):
                v = plsc.load_gather(inbuf, [lr, jnp.full((L,), s + j, jnp.int32)])
                upd = v > m
                m = jnp.where(upd, v, m)
                a = jnp.where(upd, jnp.float32(j), a)
            plsc.store_scatter(
                outbuf, [lr, jnp.full((L,), NUMERIC + slot, jnp.int32)], a
            )
        return carry

    def chunk_body(ch, carry):
        r0 = base_row + ch * CH
        pltpu.sync_copy(x_hbm.at[pl.ds(r0, CH)], inbuf)
        lax.fori_loop(0, CH // L, group_body, 0)
        pltpu.sync_copy(outbuf, out_hbm.at[pl.ds(r0, CH)])
        return carry

    lax.fori_loop(0, NCHUNK, chunk_body, 0)


def kernel(X):
    return _onehot_to_numeric(X)


# speedup vs baseline: 1.0345x; 1.0345x over previous
import functools

import jax
import jax.numpy as jnp
from jax import lax
from jax.experimental import pallas as pl
from jax.experimental.pallas import tpu as pltpu, tpu_sc as plsc

N = 262144
NCOL = 66
OCOL = 13
NUMERIC = 10
BLOCKS = ((10, 18), (18, 34), (34, 66))

NC = 2
NS = 16
L = 16
NW = NC * NS
ROWS_PER_W = N // NW
CH = 512
NCHUNK = ROWS_PER_W // CH

_mesh = plsc.VectorSubcoreMesh(core_axis_name="c", subcore_axis_name="s")


@functools.partial(
    pl.kernel,
    out_type=jax.ShapeDtypeStruct((N, OCOL), jnp.float32),
    mesh=_mesh,
    compiler_params=pltpu.CompilerParams(
        needs_layout_passes=False, use_tc_tiling_on_sc=False
    ),
    scratch_types=[
        pltpu.VMEM((CH, NCOL), jnp.float32),
        pltpu.VMEM((CH, OCOL), jnp.float32),
    ],
)
def _onehot_to_numeric(x_hbm, out_hbm, inbuf, outbuf):
    wid = lax.axis_index("s") * NC + lax.axis_index("c")
    base_row = wid * ROWS_PER_W
    lane = lax.iota(jnp.int32, L)

    def group_body(g, carry):
        lr = g * L + lane
        for c in range(NUMERIC):
            cvec = jnp.full((L,), c, jnp.int32)
            v = plsc.load_gather(inbuf, [lr, cvec])
            plsc.store_scatter(outbuf, [lr, cvec], v)
        for slot, (s, e) in enumerate(BLOCKS):
            m = plsc.load_gather(inbuf, [lr, jnp.full((L,), s, jnp.int32)])
            a = jnp.zeros((L,), jnp.float32)
            for j in range(1, e - s):
                v = plsc.load_gather(inbuf, [lr, jnp.full((L,), s + j, jnp.int32)])
                upd = v > m
                m = jnp.where(upd, v, m)
                a = jnp.where(upd, jnp.float32(j), a)
            plsc.store_scatter(
                outbuf, [lr, jnp.full((L,), NUMERIC + slot, jnp.int32)], a
            )
        return carry

    def chunk_body(ch, carry):
        r0 = base_row + ch * CH
        pltpu.sync_copy(x_hbm.at[pl.ds(r0, CH)], inbuf)
        lax.fori_loop(0, CH // L, group_body, 0)
        pltpu.sync_copy(outbuf, out_hbm.at[pl.ds(r0, CH)])
        return carry

    lax.fori_loop(0, NCHUNK, chunk_body, 0)


def kernel(X):
    return _onehot_to_numeric(X)


# speedup vs baseline: 1.1105x; 1.0734x over previous
import functools

import jax
import jax.numpy as jnp
from jax import lax
from jax.experimental import pallas as pl
from jax.experimental.pallas import tpu as pltpu, tpu_sc as plsc

N = 262144
NCOL = 66
OCOL = 13
NUMERIC = 10
BLOCKS = ((10, 18), (18, 34), (34, 66))

NC = 2
NS = 16
L = 16
NW = NC * NS
ROWS_PER_W = N // NW
CH = 512
NCHUNK = ROWS_PER_W // CH

_mesh = plsc.VectorSubcoreMesh(core_axis_name="c", subcore_axis_name="s")


@functools.partial(
    pl.kernel,
    out_type=jax.ShapeDtypeStruct((N, OCOL), jnp.float32),
    mesh=_mesh,
    compiler_params=pltpu.CompilerParams(
        needs_layout_passes=False, use_tc_tiling_on_sc=False
    ),
    scratch_types=[
        pltpu.VMEM((CH, NCOL), jnp.float32),
        pltpu.VMEM((CH, NCOL), jnp.float32),
        pltpu.VMEM((CH, OCOL), jnp.float32),
        pltpu.VMEM((CH, OCOL), jnp.float32),
        pltpu.SemaphoreType.DMA,
        pltpu.SemaphoreType.DMA,
        pltpu.SemaphoreType.DMA,
        pltpu.SemaphoreType.DMA,
    ],
)
def _onehot_to_numeric(x_hbm, out_hbm, in0, in1, ob0, ob1, si0, si1, so0, so1):
    ins = (in0, in1)
    obs = (ob0, ob1)
    sis = (si0, si1)
    sos = (so0, so1)
    wid = lax.axis_index("s") * NC + lax.axis_index("c")
    base_row = wid * ROWS_PER_W
    lane = lax.iota(jnp.int32, L)

    def compute_chunk(inbuf, outbuf):
        def group_body(g, carry):
            lr = g * L + lane
            for c in range(NUMERIC):
                cvec = jnp.full((L,), c, jnp.int32)
                v = plsc.load_gather(inbuf, [lr, cvec])
                plsc.store_scatter(outbuf, [lr, cvec], v)
            for slot, (s, e) in enumerate(BLOCKS):
                m = plsc.load_gather(inbuf, [lr, jnp.full((L,), s, jnp.int32)])
                a = jnp.zeros((L,), jnp.float32)
                for j in range(1, e - s):
                    v = plsc.load_gather(
                        inbuf, [lr, jnp.full((L,), s + j, jnp.int32)]
                    )
                    upd = v > m
                    m = jnp.where(upd, v, m)
                    a = jnp.where(upd, jnp.float32(j), a)
                plsc.store_scatter(
                    outbuf, [lr, jnp.full((L,), NUMERIC + slot, jnp.int32)], a
                )
            return carry

        lax.fori_loop(0, CH // L, group_body, 0)

    def start_in(ch, b):
        r0 = base_row + ch * CH
        pltpu.async_copy(x_hbm.at[pl.ds(r0, CH)], ins[b], sis[b])

    def start_out(ch, b):
        r0 = base_row + ch * CH
        pltpu.async_copy(obs[b], out_hbm.at[pl.ds(r0, CH)], sos[b])

    def wait_in(b):
        pltpu.make_async_copy(x_hbm.at[pl.ds(0, CH)], ins[b], sis[b]).wait()

    def wait_out(b):
        pltpu.make_async_copy(obs[b], out_hbm.at[pl.ds(0, CH)], sos[b]).wait()

    start_in(0, 0)

    @pl.loop(0, NCHUNK, step=2)
    def chunk_pair(base):
        for b in (0, 1):
            ch = base + b

            @pl.when(ch + 1 < NCHUNK)
            def _():
                start_in(ch + 1, 1 - b)

            wait_in(b)

            @pl.when(ch >= 2)
            def _():
                wait_out(b)

            compute_chunk(ins[b], obs[b])
            start_out(ch, b)

    wait_out(0)
    wait_out(1)


def kernel(X):
    return _onehot_to_numeric(X)


# speedup vs baseline: 1.1814x; 1.0638x over previous
import functools

import jax
import jax.numpy as jnp
from jax import lax
from jax.experimental import pallas as pl
from jax.experimental.pallas import tpu as pltpu, tpu_sc as plsc

N = 262144
NCOL = 66
OCOL = 13
NUMERIC = 10
BLOCKS = ((10, 18), (18, 34), (34, 66))

NC = 2
NS = 16
L = 16
NW = NC * NS
ROWS_PER_W = N // NW
CH = 512
NCHUNK = ROWS_PER_W // CH

_mesh = plsc.VectorSubcoreMesh(core_axis_name="c", subcore_axis_name="s")


@functools.partial(
    pl.kernel,
    out_type=jax.ShapeDtypeStruct((N * OCOL,), jnp.float32),
    mesh=_mesh,
    compiler_params=pltpu.CompilerParams(
        needs_layout_passes=False, use_tc_tiling_on_sc=False
    ),
    scratch_types=[
        pltpu.VMEM((CH * NCOL,), jnp.float32),
        pltpu.VMEM((CH * NCOL,), jnp.float32),
        pltpu.VMEM((CH * OCOL,), jnp.float32),
        pltpu.VMEM((CH * OCOL,), jnp.float32),
        pltpu.SemaphoreType.DMA,
        pltpu.SemaphoreType.DMA,
        pltpu.SemaphoreType.DMA,
        pltpu.SemaphoreType.DMA,
    ],
)
def _onehot_to_numeric(x_hbm, out_hbm, in0, in1, ob0, ob1, si0, si1, so0, so1):
    ins = (in0, in1)
    obs = (ob0, ob1)
    sis = (si0, si1)
    sos = (so0, so1)
    wid = lax.axis_index("s") * NC + lax.axis_index("c")
    base_row = wid * ROWS_PER_W
    lane = lax.iota(jnp.int32, L)

    def compute_chunk(inbuf, outbuf):
        def group_body(g, carry):
            lr = g * L + lane
            srow = lr * NCOL
            orow = lr * OCOL
            for c in range(NUMERIC):
                v = plsc.load_gather(inbuf, [srow + c])
                plsc.store_scatter(outbuf, [orow + c], v)
            for slot, (s, e) in enumerate(BLOCKS):
                m = plsc.load_gather(inbuf, [srow + s])
                a = jnp.zeros((L,), jnp.float32)
                for j in range(1, e - s):
                    v = plsc.load_gather(inbuf, [srow + s + j])
                    upd = v > m
                    m = jnp.where(upd, v, m)
                    a = jnp.where(upd, jnp.float32(j), a)
                plsc.store_scatter(outbuf, [orow + NUMERIC + slot], a)
            return carry

        lax.fori_loop(0, CH // L, group_body, 0)

    def start_in(ch, b):
        r0 = base_row + ch * CH
        pltpu.async_copy(x_hbm.at[pl.ds(r0 * NCOL, CH * NCOL)], ins[b], sis[b])

    def start_out(ch, b):
        r0 = base_row + ch * CH
        pltpu.async_copy(obs[b], out_hbm.at[pl.ds(r0 * OCOL, CH * OCOL)], sos[b])

    def wait_in(b):
        pltpu.make_async_copy(
            x_hbm.at[pl.ds(0, CH * NCOL)], ins[b], sis[b]
        ).wait()

    def wait_out(b):
        pltpu.make_async_copy(
            obs[b], out_hbm.at[pl.ds(0, CH * OCOL)], sos[b]
        ).wait()

    start_in(0, 0)

    @pl.loop(0, NCHUNK, step=2)
    def chunk_pair(base):
        for b in (0, 1):
            ch = base + b

            @pl.when(ch + 1 < NCHUNK)
            def _():
                start_in(ch + 1, 1 - b)

            wait_in(b)

            @pl.when(ch >= 2)
            def _():
                wait_out(b)

            compute_chunk(ins[b], obs[b])
            start_out(ch, b)

    wait_out(0)
    wait_out(1)


def kernel(X):
    out = _onehot_to_numeric(X.reshape(-1))
    return out.reshape(N, OCOL)
